# baseline (speedup 1.0000x reference)
import jax
import jax.numpy as jnp
from jax.experimental import pallas as pl
from jax.experimental.pallas import tpu as pltpu

_TM = 80


def _support_kernel(x_ref, w_ref, o_ref):
    o_ref[...] = jnp.dot(x_ref[...], w_ref[...],
                         preferred_element_type=jnp.float32
                         ).astype(jnp.bfloat16)


def _gcn_kernel(adj_ref, s_ref, b_ref, o_ref):
    o_ref[...] = jnp.dot(adj_ref[...].astype(jnp.bfloat16), s_ref[...],
                         preferred_element_type=jnp.float32) + b_ref[...]


def kernel(x, adj, W, b):
    n, nfeat = x.shape
    nclass = W.shape[1]

    support = pl.pallas_call(
        _support_kernel,
        grid=(5,),
        in_specs=[
            pl.BlockSpec((n // 5, nfeat), lambda i: (i, 0)),
            pl.BlockSpec((nfeat, nclass), lambda i: (0, 0)),
        ],
        out_specs=pl.BlockSpec((n // 5, nclass), lambda i: (i, 0)),
        out_shape=jax.ShapeDtypeStruct((n, nclass), jnp.bfloat16),
    )(x, W)

    nm = n // _TM
    out = pl.pallas_call(
        _gcn_kernel,
        grid=(nm,),
        in_specs=[
            pl.BlockSpec((_TM, n), lambda m: (m, 0)),
            pl.BlockSpec((n, nclass), lambda m: (0, 0)),
            pl.BlockSpec((1, nclass), lambda m: (0, 0)),
        ],
        out_specs=pl.BlockSpec((_TM, nclass), lambda m: (m, 0)),
        out_shape=jax.ShapeDtypeStruct((n, nclass), jnp.float32),
        compiler_params=pltpu.CompilerParams(
            dimension_semantics=("parallel",),
        ),
    )(adj, support, b.reshape(1, nclass))
    return out


# speedup vs baseline: 1.4091x; 1.4091x over previous
import functools

import jax
import jax.numpy as jnp
from jax.experimental import pallas as pl
from jax.experimental.pallas import tpu as pltpu

_CHUNK = 80
_NBUF = 8


def _gcn_kernel(x_ref, w_ref, b_ref, adj_hbm, o_ref, s_ref, buf_ref, sem,
                *, chunk, nbuf, nchunks):
    def start(i):
        slot = jax.lax.rem(i, nbuf)
        pltpu.make_async_copy(
            adj_hbm.at[pl.ds(i * chunk, chunk), :],
            buf_ref.at[slot],
            sem.at[slot],
        ).start()

    for i in range(nbuf):
        start(i)

    s_ref[...] = jnp.dot(
        x_ref[...].astype(jnp.bfloat16), w_ref[...].astype(jnp.bfloat16),
        preferred_element_type=jnp.float32).astype(jnp.bfloat16)

    def body(i, carry):
        slot = jax.lax.rem(i, nbuf)
        pltpu.make_async_copy(
            adj_hbm.at[pl.ds(i * chunk, chunk), :],
            buf_ref.at[slot],
            sem.at[slot],
        ).wait()
        o_ref[pl.ds(i * chunk, chunk), :] = jnp.dot(
            buf_ref[slot].astype(jnp.bfloat16), s_ref[...],
            preferred_element_type=jnp.float32) + b_ref[...]

        @pl.when(i + nbuf < nchunks)
        def _():
            start(i + nbuf)

        return carry

    jax.lax.fori_loop(0, nchunks, body, 0)


def kernel(x, adj, W, b):
    n, nfeat = x.shape
    nclass = W.shape[1]
    nchunks = n // _CHUNK

    out = pl.pallas_call(
        functools.partial(_gcn_kernel, chunk=_CHUNK, nbuf=_NBUF,
                          nchunks=nchunks),
        in_specs=[
            pl.BlockSpec((n, nfeat), lambda: (0, 0)),
            pl.BlockSpec((nfeat, nclass), lambda: (0, 0)),
            pl.BlockSpec((1, nclass), lambda: (0, 0)),
            pl.BlockSpec(memory_space=pl.ANY),
        ],
        out_specs=pl.BlockSpec((n, nclass), lambda: (0, 0)),
        out_shape=jax.ShapeDtypeStruct((n, nclass), jnp.float32),
        scratch_shapes=[
            pltpu.VMEM((n, nclass), jnp.bfloat16),
            pltpu.VMEM((_NBUF, _CHUNK, n), jnp.float32),
            pltpu.SemaphoreType.DMA((_NBUF,)),
        ],
    )(x, W, b.reshape(1, nclass), adj)
    return out
